# baseline (speedup 1.0000x reference)
import functools

import jax
import jax.numpy as jnp
from jax import lax
from jax.experimental import pallas as pl

B = 4096
BLK = 1024
N_COINS = 250
COIN_DIM = 32
REG_DIM = 16
N_ACC = 4
N_TEMP = 40
EH = 256
EO = 128
NE = 8
FEAT_DIM = 256
PART_NAMES = ("price", "volume", "orderflow", "derived")

_SQRT2 = 1.4142135623730951
_RSQRT_EO = 1.0 / (EO ** 0.5)


def _gelu(x):
    return 0.5 * x * (1.0 + lax.erf(x / _SQRT2))


def _ln(x, g, b, eps=1e-5):
    m = jnp.mean(x, axis=-1, keepdims=True)
    xc = x - m
    v = jnp.mean(xc * xc, axis=-1, keepdims=True)
    return xc * lax.rsqrt(v + eps) * g + b


def _dot(x, w):
    return jnp.dot(x, w, preferred_element_type=jnp.float32)


def _flatten_params(p):
    f32 = jnp.float32
    w = {}
    w["coin_emb"] = jnp.zeros((256, COIN_DIM), f32).at[:N_COINS].set(
        p["coin_emb"])
    w["regime_emb"] = jnp.zeros((8, REG_DIM), f32).at[:4].set(p["regime_emb"])
    w["temp1_w"] = p["temp1"]["w"]
    w["temp1_b"] = p["temp1"]["b"][None]
    w["temp2_w"] = p["temp2"]["w"]
    w["temp2_b"] = p["temp2"]["b"][None]
    w["temp_lng"] = p["temp_lng"][None]
    w["temp_lnb"] = p["temp_lnb"][None]
    for i, name in enumerate(PART_NAMES):
        ep = p["feat_experts"][name]
        w[f"fe{i}_w1"] = ep["w1"]
        w[f"fe{i}_b1"] = ep["b1"][None]
        w[f"fe{i}_w2"] = ep["w2"]
        w[f"fe{i}_b2"] = ep["b2"][None]
        w[f"fe{i}_w3"] = ep["w3"]
        w[f"fe{i}_b3"] = ep["b3"][None]
        w[f"fe{i}_wr"] = ep["wr"]
        w[f"fe{i}_br"] = ep["br"][None]
        w[f"fe{i}_lng"] = ep["lng"][None]
        w[f"fe{i}_lnb"] = ep["lnb"][None]
        w[f"gk{i}_w"] = p["gate_keys"][name]["w"]
        w[f"gk{i}_b"] = p["gate_keys"][name]["b"][None]
    cw = p["context"]["w"]
    w["ctx_wa"] = cw[0:N_ACC]
    w["ctx_wc"] = cw[N_ACC:N_ACC + COIN_DIM]
    w["ctx_wr"] = cw[N_ACC + COIN_DIM:N_ACC + COIN_DIM + REG_DIM]
    w["ctx_wt"] = cw[N_ACC + COIN_DIM + REG_DIM:]
    w["ctx_b"] = p["context"]["b"][None]
    qw = p["gate_q"]["w"]
    w["gq_cat"] = qw[:4 * EO]
    w["gq_ctx"] = qw[4 * EO:]
    w["gq_b"] = p["gate_q"]["b"][None]
    rw = p["router1"]["w"]
    w["r1_g"] = rw[:EO]
    w["r1_r"] = rw[EO:]
    w["r1_b"] = p["router1"]["b"][None]
    w["r2_w"] = p["router2"]["w"]
    w["r2_b"] = p["router2"]["b"][None]
    for e in range(NE):
        ep = p["moe_experts"][e]
        w[f"moe{e}_w1"] = ep["w1"]
        w[f"moe{e}_b1"] = ep["b1"][None]
        w[f"moe{e}_w2"] = ep["w2"]
        w[f"moe{e}_b2"] = ep["b2"][None]
        w[f"moe{e}_w3"] = ep["w3"]
        w[f"moe{e}_b3"] = ep["b3"][None]
        w[f"moe{e}_lng"] = ep["lng"][None]
        w[f"moe{e}_lnb"] = ep["lnb"][None]
    fw = p["fus1"]["w"]
    w["f1_m"] = fw[:EO]
    w["f1_c"] = fw[EO:]
    w["f1_b"] = p["fus1"]["b"][None]
    w["f_ln1g"] = p["fus_ln1g"][None]
    w["f_ln1b"] = p["fus_ln1b"][None]
    w["f2_w"] = p["fus2"]["w"]
    w["f2_b"] = p["fus2"]["b"][None]
    w["f_ln2g"] = p["fus_ln2g"][None]
    w["f_ln2b"] = p["fus_ln2b"][None]
    for g in range(4):
        hp = p["heads"][g]
        for hname in ("lab", "mae", "mfe"):
            for lyr in ("1", "2"):
                w[f"hd_{hname}{lyr}_{g}_w"] = hp[hname + lyr]["w"]
                w[f"hd_{hname}{lyr}_{g}_b"] = hp[hname + lyr]["b"][None]
    for nm in ("conf1", "conf2", "lev1", "lev2"):
        w[nm + "_w"] = p[nm]["w"]
        w[nm + "_b"] = p[nm]["b"][None]
    return w


def _body(names, *refs):
    feats_ref, coin_ref, reg_ref, acct_ref, temp_ref = refs[:5]
    out_ref = refs[-1]
    w = {n: r for n, r in zip(names, refs[5:-1])}

    feats = feats_ref[...]
    coin_id = coin_ref[...]
    regime_id = reg_ref[...]
    acct = acct_ref[...]
    temporal = temp_ref[...]

    iota_c = lax.broadcasted_iota(jnp.int32, (BLK, 256), 1)
    oh_c = (iota_c == coin_id).astype(jnp.float32)
    coin_emb = _dot(oh_c, w["coin_emb"][...])
    iota_r = lax.broadcasted_iota(jnp.int32, (BLK, 8), 1)
    oh_r = (iota_r == regime_id).astype(jnp.float32)
    regime_emb = _dot(oh_r, w["regime_emb"][...])

    t = _gelu(_dot(temporal, w["temp1_w"][...]) + w["temp1_b"][...])
    t = _dot(t, w["temp2_w"][...]) + w["temp2_b"][...]
    temporal_enc = _ln(t, w["temp_lng"][...], w["temp_lnb"][...])

    feat_outs = []
    for i in range(4):
        x = feats[:, i * 64:(i + 1) * 64]
        h = _gelu(_dot(x, w[f"fe{i}_w1"][...]) + w[f"fe{i}_b1"][...])
        h = _gelu(_dot(h, w[f"fe{i}_w2"][...]) + w[f"fe{i}_b2"][...])
        h = _dot(h, w[f"fe{i}_w3"][...]) + w[f"fe{i}_b3"][...]
        res = _dot(x, w[f"fe{i}_wr"][...]) + w[f"fe{i}_br"][...]
        feat_outs.append(_ln(h + res, w[f"fe{i}_lng"][...],
                             w[f"fe{i}_lnb"][...]))

    ctx = (_dot(acct, w["ctx_wa"][...]) + _dot(coin_emb, w["ctx_wc"][...])
           + _dot(regime_emb, w["ctx_wr"][...])
           + _dot(temporal_enc, w["ctx_wt"][...]) + w["ctx_b"][...])
    context_enc = _gelu(ctx)

    fcat = jnp.concatenate(feat_outs, axis=-1)
    q = (w["gq_b"][...] + _dot(context_enc, w["gq_ctx"][...])
         + _dot(fcat, w["gq_cat"][...]))
    scores = []
    for i in range(4):
        k = _dot(feat_outs[i], w[f"gk{i}_w"][...]) + w[f"gk{i}_b"][...]
        scores.append(jnp.sum(q * k, axis=-1, keepdims=True) * _RSQRT_EO)
    smax = jnp.maximum(jnp.maximum(scores[0], scores[1]),
                       jnp.maximum(scores[2], scores[3]))
    exps = [jnp.exp(s - smax) for s in scores]
    denom = exps[0] + exps[1] + exps[2] + exps[3]
    gated = jnp.zeros((BLK, EO), jnp.float32)
    for i in range(4):
        gated = gated + (exps[i] / denom) * feat_outs[i]

    rh = _gelu(_dot(gated, w["r1_g"][...]) + _dot(regime_emb, w["r1_r"][...])
               + w["r1_b"][...])
    logits = _dot(rh, w["r2_w"][...]) + w["r2_b"][...]
    iota8 = lax.broadcasted_iota(jnp.int32, (BLK, NE), 1)
    m1 = jnp.max(logits, axis=-1, keepdims=True)
    i1 = jnp.min(jnp.where(logits == m1, iota8, NE), axis=-1, keepdims=True)
    masked = jnp.where(iota8 == i1, -1e30, logits)
    m2 = jnp.max(masked, axis=-1, keepdims=True)
    i2 = jnp.min(jnp.where(masked == m2, iota8, NE), axis=-1, keepdims=True)
    e2 = jnp.exp(m2 - m1)
    w1c = 1.0 / (1.0 + e2)
    w2c = e2 * w1c
    coefs = (jnp.where(iota8 == i1, w1c, 0.0)
             + jnp.where(iota8 == i2, w2c, 0.0))

    moe = jnp.zeros((BLK, EO), jnp.float32)
    for e in range(NE):
        h = _gelu(_dot(gated, w[f"moe{e}_w1"][...]) + w[f"moe{e}_b1"][...])
        h = _gelu(_dot(h, w[f"moe{e}_w2"][...]) + w[f"moe{e}_b2"][...])
        h = _dot(h, w[f"moe{e}_w3"][...]) + w[f"moe{e}_b3"][...]
        eo = _ln(h + gated, w[f"moe{e}_lng"][...], w[f"moe{e}_lnb"][...])
        moe = moe + lax.slice_in_dim(coefs, e, e + 1, axis=1) * eo

    f = _gelu(_dot(moe, w["f1_m"][...]) + _dot(context_enc, w["f1_c"][...])
              + w["f1_b"][...])
    f = _ln(f, w["f_ln1g"][...], w["f_ln1b"][...])
    f = _gelu(_dot(f, w["f2_w"][...]) + w["f2_b"][...])
    f = _ln(f, w["f_ln2g"][...], w["f_ln2b"][...])

    pieces = []
    for hname in ("lab", "mae", "mfe"):
        for g in range(4):
            h1 = _gelu(_dot(f, w[f"hd_{hname}1_{g}_w"][...])
                       + w[f"hd_{hname}1_{g}_b"][...])
            pieces.append(_dot(h1, w[f"hd_{hname}2_{g}_w"][...])
                          + w[f"hd_{hname}2_{g}_b"][...])
    c = _gelu(_dot(f, w["conf1_w"][...]) + w["conf1_b"][...])
    pieces.append(jax.nn.sigmoid(_dot(c, w["conf2_w"][...])
                                 + w["conf2_b"][...]))
    lv = _gelu(_dot(f, w["lev1_w"][...]) + w["lev1_b"][...])
    pieces.append(jax.nn.sigmoid(_dot(lv, w["lev2_w"][...])
                                 + w["lev2_b"][...]))
    out_ref[...] = jnp.concatenate(pieces, axis=-1)


def _forward(features, coin_id, regime_id, account, temporal, params,
             interpret=False):
    w = _flatten_params(params)
    names = tuple(w.keys())
    warrs = [w[n] for n in names]
    coin2 = coin_id.astype(jnp.int32).reshape(B, 1)
    reg2 = regime_id.astype(jnp.int32).reshape(B, 1)

    def _const_spec(arr):
        nd = arr.ndim
        return pl.BlockSpec(arr.shape, lambda i, _nd=nd: (0,) * _nd)

    in_specs = [
        pl.BlockSpec((BLK, FEAT_DIM), lambda i: (i, 0)),
        pl.BlockSpec((BLK, 1), lambda i: (i, 0)),
        pl.BlockSpec((BLK, 1), lambda i: (i, 0)),
        pl.BlockSpec((BLK, N_ACC), lambda i: (i, 0)),
        pl.BlockSpec((BLK, N_TEMP), lambda i: (i, 0)),
    ] + [_const_spec(a) for a in warrs]

    out = pl.pallas_call(
        functools.partial(_body, names),
        grid=(B // BLK,),
        in_specs=in_specs,
        out_specs=pl.BlockSpec((BLK, 98), lambda i: (i, 0)),
        out_shape=jax.ShapeDtypeStruct((B, 98), jnp.float32),
        interpret=interpret,
    )(features, coin2, reg2, account, temporal, *warrs)
    return out


def kernel(features, coin_id, regime_id, account, temporal, params):
    return _forward(features, coin_id, regime_id, account, temporal, params)
